# baseline (speedup 1.0000x reference)
import functools

import jax
import jax.numpy as jnp
from jax import lax
from jax.experimental import pallas as pl
from jax.experimental.pallas import tpu as pltpu
from jax.experimental.pallas import tpu_sc as plsc

B = 4
S = 8192
D = 768
LANES = 16
NVEC = D // LANES
BLK = 8
NBLK = NVEC // BLK

NC = 2
NS = 16
NW = NC * NS
S_PER_W = S // NW
C = 8
R = B * C
NITEM = S_PER_W // C
NBUF = 4

_mesh = plsc.VectorSubcoreMesh(core_axis_name="c", subcore_axis_name="s")


@functools.partial(
    pl.kernel,
    mesh=_mesh,
    out_type=jax.ShapeDtypeStruct((B, S, D), jnp.float32),
    scratch_types=[
        pltpu.VMEM((NITEM, R), jnp.int32),
        pltpu.VMEM((NBUF, R, D), jnp.float32),
        pltpu.VMEM((NBUF, C, D), jnp.float32),
        pltpu.SemaphoreType.DMA,
    ] + [pltpu.SemaphoreType.DMA] * (3 * NBUF),
)
def _emb_lookup(x_hbm, emb_hbm, pos_hbm, out_hbm, idx_items, embs, pos_v,
                isem, *sems):
    gsems = sems[0:NBUF]
    osems = sems[NBUF:2 * NBUF]
    psems = sems[2 * NBUF:3 * NBUF]
    wid = lax.axis_index("s") * NC + lax.axis_index("c")
    s_base = wid * S_PER_W

    for t in range(NITEM):
        for b in range(B):
            pltpu.async_copy(
                x_hbm.at[b, pl.ds(s_base + t * C, C)],
                idx_items.at[t, pl.ds(b * C, C)], isem)
    for t in range(NITEM):
        for b in range(B):
            pltpu.make_async_copy(
                x_hbm.at[b, pl.ds(s_base + t * C, C)],
                idx_items.at[t, pl.ds(b * C, C)], isem).wait()

    def start_item(t):
        for k in range(NBUF):
            @pl.when(lax.rem(t, NBUF) == k)
            def _():
                pltpu.async_copy(
                    pos_hbm.at[pl.ds(s_base + t * C, C), :], pos_v.at[k], psems[k])
                pltpu.async_copy(
                    emb_hbm.at[idx_items.at[t]], embs.at[k], gsems[k])

    def wait_item(t):
        for k in range(NBUF):
            @pl.when(lax.rem(t, NBUF) == k)
            def _():
                pltpu.make_async_copy(
                    pos_hbm.at[pl.ds(s_base + t * C, C), :], pos_v.at[k], psems[k]
                ).wait()
                pltpu.make_async_copy(
                    emb_hbm.at[idx_items.at[t]], embs.at[k], gsems[k]).wait()

    def start_out(t):
        s0 = s_base + t * C
        for k in range(NBUF):
            @pl.when(lax.rem(t, NBUF) == k)
            def _():
                for b in range(B):
                    pltpu.async_copy(
                        embs.at[k, pl.ds(b * C, C)],
                        out_hbm.at[b, pl.ds(s0, C), :], osems[k])

    def wait_out(t):
        s0 = s_base + t * C
        for k in range(NBUF):
            @pl.when(lax.rem(t, NBUF) == k)
            def _():
                for b in range(B):
                    pltpu.make_async_copy(
                        embs.at[k, pl.ds(b * C, C)],
                        out_hbm.at[b, pl.ds(s0, C), :], osems[k]).wait()

    def add_item(t):
        par = lax.rem(t, NBUF)

        def row_body(r, _):
            for jb in range(NBLK):
                base = jb * BLK * LANES
                ps = [pos_v[par, r, pl.ds(base + j * LANES, LANES)]
                      for j in range(BLK)]
                for b in range(B):
                    es = [embs[par, b * C + r, pl.ds(base + j * LANES, LANES)]
                          for j in range(BLK)]
                    ss = [e + p for e, p in zip(es, ps)]
                    for j in range(BLK):
                        embs[par, b * C + r, pl.ds(base + j * LANES, LANES)] = ss[j]
            return 0

        lax.fori_loop(0, C, row_body, 0)

    start_item(0)
    start_item(1)

    def pipe_body(i, _):
        t_c = i - 2

        @pl.when(i < NITEM)
        def _():
            @pl.when(i >= NBUF)
            def _():
                wait_out(i - NBUF)

            start_item(i)

        wait_item(t_c)
        add_item(t_c)
        start_out(t_c)
        return 0

    lax.fori_loop(2, NITEM + 2, pipe_body, 0)

    for t in range(NITEM - NBUF, NITEM):
        wait_out(t)


def kernel(x, emb_table, pos_table):
    return _emb_lookup(x.astype(jnp.int32), emb_table, pos_table)
